# Initial kernel scaffold; baseline (speedup 1.0000x reference)
import functools

import jax
import jax.numpy as jnp
from jax import lax
from jax.experimental import pallas as pl
from jax.experimental.pallas import tpu as pltpu
from jax.experimental.pallas import tpu_sc as plsc

N_NODES = 10000
NP = 10240
D = 128
E_REAL = 320000
EP_DEG = 327680
E_MSG = E_REAL + NP
EP_MSG = 331776
NC = 2
NS = 16
NW = NC * NS
L = 16
CHUNK = 128
DEG_CH = EP_DEG // NW // CHUNK
MSG_CH = EP_MSG // NW // CHUNK
ROWS_T = NP // NS
MASK_P = 1024
MPW = MASK_P // NW

_mesh = plsc.VectorSubcoreMesh(core_axis_name="c", subcore_axis_name="s")


def _wid():
    return lax.axis_index("s") * NC + lax.axis_index("c")


@functools.partial(
    pl.kernel,
    out_type=jax.ShapeDtypeStruct((NC, NP), jnp.float32),
    mesh=_mesh,
    scratch_types=[
        pltpu.VMEM((DEG_CH, CHUNK), jnp.int32),
        pltpu.VMEM((DEG_CH, CHUNK), jnp.float32),
        pltpu.VMEM((ROWS_T,), jnp.float32),
        pltpu.VMEM_SHARED((NP,), jnp.float32),
    ],
)
def _deg_kernel(dst_hbm, ew_hbm, degp_hbm, dst_s, ew_s, zb, dacc):
    cid = lax.axis_index("c")
    tid = lax.axis_index("s")
    wid = _wid()
    pltpu.sync_copy(dst_hbm.at[wid], dst_s)
    pltpu.sync_copy(ew_hbm.at[wid], ew_s)

    def zbody(r, _):
        zb[pl.ds(r * L, L)] = jnp.zeros((L,), jnp.float32)
        return 0
    lax.fori_loop(0, ROWS_T // L, zbody, 0)
    pltpu.sync_copy(zb, dacc.at[pl.ds(tid * ROWS_T, ROWS_T)])
    plsc.subcore_barrier()

    def body(k, _):
        pltpu.sync_copy(ew_s.at[k], dacc.at[dst_s.at[k]], add=True)
        return 0
    lax.fori_loop(0, DEG_CH, body, 0)
    plsc.subcore_barrier()
    pltpu.sync_copy(dacc.at[pl.ds(tid * ROWS_T, ROWS_T)],
                    degp_hbm.at[cid, pl.ds(tid * ROWS_T, ROWS_T)])


def _dinv_body(d0_ref, d1_ref, o_ref):
    deg = d0_ref[...] + d1_ref[...] + 1.0
    o_ref[...] = lax.rsqrt(deg)


def _dinv_call(d0, d1):
    return pl.pallas_call(
        _dinv_body,
        out_shape=jax.ShapeDtypeStruct((NP // D, D), jnp.float32),
    )(d0, d1)


def _mm_body(x_ref, w_ref, o_ref):
    o_ref[...] = jnp.dot(x_ref[...], w_ref[...],
                         preferred_element_type=jnp.float32)


def _mm_call(x, w):
    bm = 512
    return pl.pallas_call(
        _mm_body,
        grid=(NP // bm,),
        in_specs=[pl.BlockSpec((bm, D), lambda i: (i, 0)),
                  pl.BlockSpec((D, D), lambda i: (0, 0))],
        out_specs=pl.BlockSpec((bm, D), lambda i: (i, 0)),
        out_shape=jax.ShapeDtypeStruct((NP, D), jnp.float32),
    )(x, w)


def _layer2_body(a0_ref, a1_ref, b_ref, w_ref, o_ref):
    z = a0_ref[...] + a1_ref[...] + b_ref[0:1, :]
    z = jnp.maximum(z, 0.0)
    o_ref[...] = jnp.dot(z, w_ref[...], preferred_element_type=jnp.float32)


def _layer2_call(a0, a1, b8, w):
    bm = 512
    return pl.pallas_call(
        _layer2_body,
        grid=(NP // bm,),
        in_specs=[pl.BlockSpec((bm, D), lambda i: (i, 0)),
                  pl.BlockSpec((bm, D), lambda i: (i, 0)),
                  pl.BlockSpec((8, D), lambda i: (0, 0)),
                  pl.BlockSpec((D, D), lambda i: (0, 0))],
        out_specs=pl.BlockSpec((bm, D), lambda i: (i, 0)),
        out_shape=jax.ShapeDtypeStruct((NP, D), jnp.float32),
    )(a0, a1, b8, w)


@functools.partial(
    pl.kernel,
    out_type=(jax.ShapeDtypeStruct((NP, D), jnp.float32),
              jax.ShapeDtypeStruct((NP, D), jnp.float32)),
    mesh=_mesh,
    scratch_types=[
        pltpu.VMEM((MSG_CH, CHUNK), jnp.int32),
        pltpu.VMEM((MSG_CH, CHUNK), jnp.int32),
        pltpu.VMEM((MSG_CH, CHUNK), jnp.float32),
        pltpu.VMEM((CHUNK,), jnp.float32),
        pltpu.VMEM((CHUNK,), jnp.float32),
        pltpu.VMEM((CHUNK,), jnp.float32),
        pltpu.VMEM((CHUNK, D), jnp.float32),
        pltpu.VMEM_SHARED((NP, D), jnp.float32),
        pltpu.SemaphoreType.DMA,
    ],
)
def _msg_kernel(h_hbm, src_hbm, dst_hbm, ew_hbm, dinv_hbm,
                agg0_hbm, agg1_hbm,
                src_s, dst_s, ew_s, ds_s, dd_s, s_s, rows, acc, sem):
    cid = lax.axis_index("c")
    tid = lax.axis_index("s")
    wid = _wid()
    pltpu.sync_copy(src_hbm.at[wid], src_s)
    pltpu.sync_copy(dst_hbm.at[wid], dst_s)
    pltpu.sync_copy(ew_hbm.at[wid], ew_s)

    def zrow(r, _):
        for j in range(D // L):
            rows[r, pl.ds(j * L, L)] = jnp.zeros((L,), jnp.float32)
        return 0
    lax.fori_loop(0, CHUNK, zrow, 0)
    for i in range(ROWS_T // CHUNK):
        pltpu.sync_copy(rows, acc.at[pl.ds(tid * ROWS_T + i * CHUNK, CHUNK)])
    plsc.subcore_barrier()

    def body(k, _):
        g1 = pltpu.async_copy(h_hbm.at[src_s.at[k]], rows, sem)
        g2 = pltpu.async_copy(dinv_hbm.at[src_s.at[k]], ds_s, sem)
        g3 = pltpu.async_copy(dinv_hbm.at[dst_s.at[k]], dd_s, sem)
        g1.wait()
        g2.wait()
        g3.wait()
        for j in range(CHUNK // L):
            sl = pl.ds(j * L, L)
            s_s[sl] = ew_s[k, sl] * ds_s[sl] * dd_s[sl]

        def scale(r, _):
            sc = lax.broadcast(s_s[r], (L,))
            for j in range(D // L):
                sl = pl.ds(j * L, L)
                rows[r, sl] = rows[r, sl] * sc
            return 0
        lax.fori_loop(0, CHUNK, scale, 0)
        pltpu.sync_copy(rows, acc.at[dst_s.at[k]], add=True)
        return 0
    lax.fori_loop(0, MSG_CH, body, 0)
    plsc.subcore_barrier()

    @pl.when(cid == 0)
    def _():
        for i in range(ROWS_T // CHUNK):
            r0 = tid * ROWS_T + i * CHUNK
            pltpu.sync_copy(acc.at[pl.ds(r0, CHUNK)],
                            agg0_hbm.at[pl.ds(r0, CHUNK)])

    @pl.when(cid == 1)
    def _():
        for i in range(ROWS_T // CHUNK):
            r0 = tid * ROWS_T + i * CHUNK
            pltpu.sync_copy(acc.at[pl.ds(r0, CHUNK)],
                            agg1_hbm.at[pl.ds(r0, CHUNK)])


@functools.partial(
    pl.kernel,
    out_type=(jax.ShapeDtypeStruct((MASK_P, D), jnp.float32),
              jax.ShapeDtypeStruct((MASK_P,), jnp.int32)),
    mesh=_mesh,
    scratch_types=[
        pltpu.VMEM((MPW,), jnp.int32),
        pltpu.VMEM((MPW, D), jnp.float32),
        pltpu.VMEM((MPW, D), jnp.float32),
        pltpu.VMEM((MPW,), jnp.int32),
        pltpu.VMEM((D,), jnp.float32),
        pltpu.SemaphoreType.DMA,
    ],
)
def _final_kernel(a0_hbm, a1_hbm, b2_hbm, mask_hbm, y_hbm,
                  out_hbm, ym_hbm,
                  m_s, r0, r1, yb, b2_s, sem):
    wid = _wid()
    pltpu.sync_copy(mask_hbm.at[wid], m_s)
    pltpu.sync_copy(b2_hbm, b2_s)
    g1 = pltpu.async_copy(a0_hbm.at[m_s], r0, sem)
    g2 = pltpu.async_copy(a1_hbm.at[m_s], r1, sem)
    g3 = pltpu.async_copy(y_hbm.at[m_s], yb, sem)
    g1.wait()
    g2.wait()
    g3.wait()

    def body(r, _):
        for j in range(D // L):
            sl = pl.ds(j * L, L)
            r0[r, sl] = r0[r, sl] + r1[r, sl] + b2_s[sl]
        return 0
    lax.fori_loop(0, MPW, body, 0)
    pltpu.sync_copy(r0, out_hbm.at[pl.ds(wid * MPW, MPW)])
    pltpu.sync_copy(yb, ym_hbm.at[pl.ds(wid * MPW, MPW)])


def kernel(x, edge_index, edge_weight, mask_idx, y, W1, b1, W2, b2):
    src = edge_index[0].astype(jnp.int32)
    dst = edge_index[1].astype(jnp.int32)
    ew = edge_weight.astype(jnp.float32)

    dstp = jnp.pad(dst, (0, EP_DEG - E_REAL)).reshape(NW, DEG_CH, CHUNK)
    ewp = jnp.pad(ew, (0, EP_DEG - E_REAL)).reshape(NW, DEG_CH, CHUNK)

    loop_idx = jnp.arange(NP, dtype=jnp.int32)
    src2 = jnp.pad(jnp.concatenate([src, loop_idx]),
                   (0, EP_MSG - E_MSG)).reshape(NW, MSG_CH, CHUNK)
    dst2 = jnp.pad(jnp.concatenate([dst, loop_idx]),
                   (0, EP_MSG - E_MSG)).reshape(NW, MSG_CH, CHUNK)
    ew2 = jnp.pad(jnp.concatenate([ew, jnp.ones((NP,), jnp.float32)]),
                  (0, EP_MSG - E_MSG)).reshape(NW, MSG_CH, CHUNK)

    x_p = jnp.pad(x, ((0, NP - N_NODES), (0, 0)))
    mask_p = jnp.pad(mask_idx.astype(jnp.int32),
                     (0, MASK_P - mask_idx.shape[0])).reshape(NW, MPW)
    b1_8 = jnp.broadcast_to(b1.reshape(1, D), (8, D))

    degp = _deg_kernel(dstp, ewp)
    d3 = degp.reshape(NC, NP // D, D)
    dinv2d = _dinv_call(d3[0], d3[1])
    dinv_flat = dinv2d.reshape(NP)

    h1 = _mm_call(x_p, W1)
    a0, a1 = _msg_kernel(h1, src2, dst2, ew2, dinv_flat)
    h2 = _layer2_call(a0, a1, b1_8, W2)
    c0, c1 = _msg_kernel(h2, src2, dst2, ew2, dinv_flat)
    out_p, ym_p = _final_kernel(c0, c1, b2, mask_p, y.astype(jnp.int32))
    n_mask = mask_idx.shape[0]
    return out_p[:n_mask], ym_p[:n_mask]


# speedup vs baseline: 3.7769x; 3.7769x over previous
import functools

import jax
import jax.numpy as jnp
from jax import lax
from jax.experimental import pallas as pl
from jax.experimental.pallas import tpu as pltpu
from jax.experimental.pallas import tpu_sc as plsc

N_NODES = 10000
NP = 10240
D = 128
E_REAL = 320000
EP_DEG = 327680
E_MSG = E_REAL + NP
EP_MSG = 360448
NC = 2
NS = 16
NW = NC * NS
L = 16
CHUNK = 128
DEG_CH = EP_DEG // NW // CHUNK
MSG_CH = EP_MSG // NW // CHUNK
MSG_G = 8
ROWS_T = NP // NS
MASK_P = 1024
MPW = MASK_P // NW

_mesh = plsc.VectorSubcoreMesh(core_axis_name="c", subcore_axis_name="s")


def _wid():
    return lax.axis_index("s") * NC + lax.axis_index("c")


@functools.partial(
    pl.kernel,
    out_type=(jax.ShapeDtypeStruct((NP,), jnp.float32),
              jax.ShapeDtypeStruct((NP,), jnp.float32)),
    mesh=_mesh,
    scratch_types=[
        pltpu.VMEM((DEG_CH, CHUNK), jnp.int32),
        pltpu.VMEM((DEG_CH, CHUNK), jnp.float32),
        pltpu.VMEM((ROWS_T,), jnp.float32),
        pltpu.VMEM_SHARED((NP,), jnp.float32),
    ],
)
def _deg_kernel(dst_hbm, ew_hbm, degp0_hbm, degp1_hbm, dst_s, ew_s, zb, dacc):
    cid = lax.axis_index("c")
    tid = lax.axis_index("s")
    wid = _wid()
    pltpu.sync_copy(dst_hbm.at[wid], dst_s)
    pltpu.sync_copy(ew_hbm.at[wid], ew_s)

    def zbody(r, _):
        zb[pl.ds(r * L, L)] = jnp.zeros((L,), jnp.float32)
        return 0
    lax.fori_loop(0, ROWS_T // L, zbody, 0)
    pltpu.sync_copy(zb, dacc.at[pl.ds(tid * ROWS_T, ROWS_T)])
    plsc.subcore_barrier()

    def body(k, _):
        pltpu.sync_copy(ew_s.at[k], dacc.at[dst_s.at[k]], add=True)
        return 0
    lax.fori_loop(0, DEG_CH, body, 0)
    plsc.subcore_barrier()

    @pl.when(cid == 0)
    def _():
        pltpu.sync_copy(dacc.at[pl.ds(tid * ROWS_T, ROWS_T)],
                        degp0_hbm.at[pl.ds(tid * ROWS_T, ROWS_T)])

    @pl.when(cid == 1)
    def _():
        pltpu.sync_copy(dacc.at[pl.ds(tid * ROWS_T, ROWS_T)],
                        degp1_hbm.at[pl.ds(tid * ROWS_T, ROWS_T)])


def _dinv_body(d0_ref, d1_ref, o_ref):
    deg = d0_ref[...] + d1_ref[...] + 1.0
    o_ref[...] = lax.rsqrt(deg)


def _dinv_call(d0, d1):
    return pl.pallas_call(
        _dinv_body,
        out_shape=jax.ShapeDtypeStruct((NP // D, D), jnp.float32),
    )(d0, d1)


def _mm_body(x_ref, w_ref, o_ref):
    o_ref[...] = jnp.dot(x_ref[...], w_ref[...],
                         preferred_element_type=jnp.float32)


def _mm_call(x, w):
    bm = 512
    return pl.pallas_call(
        _mm_body,
        grid=(NP // bm,),
        in_specs=[pl.BlockSpec((bm, D), lambda i: (i, 0)),
                  pl.BlockSpec((D, D), lambda i: (0, 0))],
        out_specs=pl.BlockSpec((bm, D), lambda i: (i, 0)),
        out_shape=jax.ShapeDtypeStruct((NP, D), jnp.float32),
    )(x, w)


def _layer2_body(a0_ref, a1_ref, b_ref, w_ref, o_ref):
    z = a0_ref[...] + a1_ref[...] + b_ref[0:1, :]
    z = jnp.maximum(z, 0.0)
    o_ref[...] = jnp.dot(z, w_ref[...], preferred_element_type=jnp.float32)


def _layer2_call(a0, a1, b8, w):
    bm = 512
    return pl.pallas_call(
        _layer2_body,
        grid=(NP // bm,),
        in_specs=[pl.BlockSpec((bm, D), lambda i: (i, 0)),
                  pl.BlockSpec((bm, D), lambda i: (i, 0)),
                  pl.BlockSpec((8, D), lambda i: (0, 0)),
                  pl.BlockSpec((D, D), lambda i: (0, 0))],
        out_specs=pl.BlockSpec((bm, D), lambda i: (i, 0)),
        out_shape=jax.ShapeDtypeStruct((NP, D), jnp.float32),
    )(a0, a1, b8, w)


@functools.partial(
    pl.kernel,
    out_type=(jax.ShapeDtypeStruct((NP, D), jnp.float32),
              jax.ShapeDtypeStruct((NP, D), jnp.float32)),
    mesh=_mesh,
    scratch_types=[
        pltpu.VMEM((MSG_G, CHUNK), jnp.int32),
        pltpu.VMEM((MSG_G, CHUNK), jnp.int32),
        pltpu.VMEM((MSG_G, CHUNK), jnp.float32),
        pltpu.VMEM((CHUNK,), jnp.float32),
        pltpu.VMEM((CHUNK,), jnp.float32),
        pltpu.VMEM((CHUNK,), jnp.float32),
        pltpu.VMEM((CHUNK, D), jnp.float32),
        pltpu.VMEM_SHARED((NP, D), jnp.float32),
        pltpu.SemaphoreType.DMA,
    ],
)
def _msg_kernel(h_hbm, src_hbm, dst_hbm, ew_hbm, dinv_hbm,
                agg0_hbm, agg1_hbm,
                src_s, dst_s, ew_s, ds_s, dd_s, s_s, rows, acc, sem):
    cid = lax.axis_index("c")
    tid = lax.axis_index("s")
    wid = _wid()

    def zrow(r, _):
        for j in range(D // L):
            rows[r, pl.ds(j * L, L)] = jnp.zeros((L,), jnp.float32)
        return 0
    lax.fori_loop(0, CHUNK, zrow, 0)
    for i in range(ROWS_T // CHUNK):
        pltpu.sync_copy(rows, acc.at[pl.ds(tid * ROWS_T + i * CHUNK, CHUNK)])
    plsc.subcore_barrier()

    def group(g, _):
        pltpu.sync_copy(src_hbm.at[wid, pl.ds(g * MSG_G, MSG_G)], src_s)
        pltpu.sync_copy(dst_hbm.at[wid, pl.ds(g * MSG_G, MSG_G)], dst_s)
        pltpu.sync_copy(ew_hbm.at[wid, pl.ds(g * MSG_G, MSG_G)], ew_s)

        def body(k, _):
            g1 = pltpu.async_copy(h_hbm.at[src_s.at[k]], rows, sem)
            g2 = pltpu.async_copy(dinv_hbm.at[src_s.at[k]], ds_s, sem)
            g3 = pltpu.async_copy(dinv_hbm.at[dst_s.at[k]], dd_s, sem)
            g1.wait()
            g2.wait()
            g3.wait()
            for j in range(CHUNK // L):
                sl = pl.ds(j * L, L)
                s_s[sl] = ew_s[k, sl] * ds_s[sl] * dd_s[sl]

            def scale(jg, _):
                sv = s_s[pl.ds(jg * L, L)]
                for r2 in range(L):
                    sc = lax.broadcast(sv[r2], (L,))
                    r = jg * L + r2
                    for j in range(D // L):
                        sl = pl.ds(j * L, L)
                        rows[r, sl] = rows[r, sl] * sc
                return 0
            lax.fori_loop(0, CHUNK // L, scale, 0)
            pltpu.sync_copy(rows, acc.at[dst_s.at[k]], add=True)
            return 0
        lax.fori_loop(0, MSG_G, body, 0)
        return 0
    lax.fori_loop(0, MSG_CH // MSG_G, group, 0)
    plsc.subcore_barrier()

    @pl.when(cid == 0)
    def _():
        for i in range(ROWS_T // CHUNK):
            r0 = tid * ROWS_T + i * CHUNK
            pltpu.sync_copy(acc.at[pl.ds(r0, CHUNK)],
                            agg0_hbm.at[pl.ds(r0, CHUNK)])

    @pl.when(cid == 1)
    def _():
        for i in range(ROWS_T // CHUNK):
            r0 = tid * ROWS_T + i * CHUNK
            pltpu.sync_copy(acc.at[pl.ds(r0, CHUNK)],
                            agg1_hbm.at[pl.ds(r0, CHUNK)])


@functools.partial(
    pl.kernel,
    out_type=(jax.ShapeDtypeStruct((MASK_P, D), jnp.float32),
              jax.ShapeDtypeStruct((MASK_P,), jnp.int32)),
    mesh=_mesh,
    scratch_types=[
        pltpu.VMEM((MPW,), jnp.int32),
        pltpu.VMEM((MPW, D), jnp.float32),
        pltpu.VMEM((MPW, D), jnp.float32),
        pltpu.VMEM((MPW,), jnp.int32),
        pltpu.VMEM((D,), jnp.float32),
        pltpu.SemaphoreType.DMA,
    ],
)
def _final_kernel(a0_hbm, a1_hbm, b2_hbm, mask_hbm, y_hbm,
                  out_hbm, ym_hbm,
                  m_s, r0, r1, yb, b2_s, sem):
    wid = _wid()
    pltpu.sync_copy(mask_hbm.at[wid], m_s)
    pltpu.sync_copy(b2_hbm, b2_s)
    g1 = pltpu.async_copy(a0_hbm.at[m_s], r0, sem)
    g2 = pltpu.async_copy(a1_hbm.at[m_s], r1, sem)
    g3 = pltpu.async_copy(y_hbm.at[m_s], yb, sem)
    g1.wait()
    g2.wait()
    g3.wait()

    def body(r, _):
        for j in range(D // L):
            sl = pl.ds(j * L, L)
            r0[r, sl] = r0[r, sl] + r1[r, sl] + b2_s[sl]
        return 0
    lax.fori_loop(0, MPW, body, 0)
    pltpu.sync_copy(r0, out_hbm.at[pl.ds(wid * MPW, MPW)])
    pltpu.sync_copy(yb, ym_hbm.at[pl.ds(wid * MPW, MPW)])


def kernel(x, edge_index, edge_weight, mask_idx, y, W1, b1, W2, b2):
    src = edge_index[0].astype(jnp.int32)
    dst = edge_index[1].astype(jnp.int32)
    ew = edge_weight.astype(jnp.float32)

    dstp = jnp.pad(dst, (0, EP_DEG - E_REAL)).reshape(NW, DEG_CH, CHUNK)
    ewp = jnp.pad(ew, (0, EP_DEG - E_REAL)).reshape(NW, DEG_CH, CHUNK)

    loop_idx = jnp.arange(NP, dtype=jnp.int32)
    src2 = jnp.pad(jnp.concatenate([src, loop_idx]),
                   (0, EP_MSG - E_MSG)).reshape(NW, MSG_CH, CHUNK)
    dst2 = jnp.pad(jnp.concatenate([dst, loop_idx]),
                   (0, EP_MSG - E_MSG)).reshape(NW, MSG_CH, CHUNK)
    ew2 = jnp.pad(jnp.concatenate([ew, jnp.ones((NP,), jnp.float32)]),
                  (0, EP_MSG - E_MSG)).reshape(NW, MSG_CH, CHUNK)

    x_p = jnp.pad(x, ((0, NP - N_NODES), (0, 0)))
    mask_p = jnp.pad(mask_idx.astype(jnp.int32),
                     (0, MASK_P - mask_idx.shape[0])).reshape(NW, MPW)
    b1_8 = jnp.broadcast_to(b1.reshape(1, D), (8, D))

    degp0, degp1 = _deg_kernel(dstp, ewp)
    dinv2d = _dinv_call(degp0.reshape(NP // D, D), degp1.reshape(NP // D, D))
    dinv_flat = dinv2d.reshape(NP)

    h1 = _mm_call(x_p, W1)
    a0, a1 = _msg_kernel(h1, src2, dst2, ew2, dinv_flat)
    h2 = _layer2_call(a0, a1, b1_8, W2)
    c0, c1 = _msg_kernel(h2, src2, dst2, ew2, dinv_flat)
    out_p, ym_p = _final_kernel(c0, c1, b2, mask_p, y.astype(jnp.int32))
    n_mask = mask_idx.shape[0]
    return out_p[:n_mask], ym_p[:n_mask]


# speedup vs baseline: 8.6366x; 2.2867x over previous
import functools

import jax
import jax.numpy as jnp
from jax import lax
from jax.experimental import pallas as pl
from jax.experimental.pallas import tpu as pltpu
from jax.experimental.pallas import tpu_sc as plsc

N_NODES = 10000
NP = 10240
D = 128
E_REAL = 320000
EP = 327680
NC = 2
NS = 16
NW = NC * NS
L = 16
CHUNK = 128
NCH = EP // NW // CHUNK
G = 8
NG = NCH // G
ROWS_T = NP // NS
MASK_P = 1024
MPW = MASK_P // NW

_mesh = plsc.VectorSubcoreMesh(core_axis_name="c", subcore_axis_name="s")


def _wid():
    return lax.axis_index("s") * NC + lax.axis_index("c")


@functools.partial(
    pl.kernel,
    out_type=(jax.ShapeDtypeStruct((NP,), jnp.float32),
              jax.ShapeDtypeStruct((NP,), jnp.float32)),
    mesh=_mesh,
    scratch_types=[
        pltpu.VMEM((NCH, CHUNK), jnp.int32),
        pltpu.VMEM((NCH, CHUNK), jnp.float32),
        pltpu.VMEM((ROWS_T,), jnp.float32),
        pltpu.VMEM_SHARED((NP,), jnp.float32),
    ],
)
def _deg_kernel(dst_hbm, ew_hbm, degp0_hbm, degp1_hbm, dst_s, ew_s, zb, dacc):
    cid = lax.axis_index("c")
    tid = lax.axis_index("s")
    wid = _wid()
    pltpu.sync_copy(dst_hbm.at[wid], dst_s)
    pltpu.sync_copy(ew_hbm.at[wid], ew_s)

    def zbody(r, _):
        zb[pl.ds(r * L, L)] = jnp.zeros((L,), jnp.float32)
        return 0
    lax.fori_loop(0, ROWS_T // L, zbody, 0)
    pltpu.sync_copy(zb, dacc.at[pl.ds(tid * ROWS_T, ROWS_T)])
    plsc.subcore_barrier()

    def body(k, _):
        pltpu.sync_copy(ew_s.at[k], dacc.at[dst_s.at[k]], add=True)
        return 0
    lax.fori_loop(0, NCH, body, 0)
    plsc.subcore_barrier()

    @pl.when(cid == 0)
    def _():
        pltpu.sync_copy(dacc.at[pl.ds(tid * ROWS_T, ROWS_T)],
                        degp0_hbm.at[pl.ds(tid * ROWS_T, ROWS_T)])

    @pl.when(cid == 1)
    def _():
        pltpu.sync_copy(dacc.at[pl.ds(tid * ROWS_T, ROWS_T)],
                        degp1_hbm.at[pl.ds(tid * ROWS_T, ROWS_T)])


def _dinv_body(d0_ref, d1_ref, o_ref):
    deg = d0_ref[...] + d1_ref[...] + 1.0
    o_ref[...] = lax.rsqrt(deg)


def _dinv_call(d0, d1):
    return pl.pallas_call(
        _dinv_body,
        out_shape=jax.ShapeDtypeStruct((NP // D, D), jnp.float32),
    )(d0, d1)


def _mm_body(x_ref, w_ref, d2_ref, o_ref):
    o_ref[...] = jnp.dot(x_ref[...], w_ref[...],
                         preferred_element_type=jnp.float32) * d2_ref[...]


def _mm_call(x, w, d2):
    bm = 512
    return pl.pallas_call(
        _mm_body,
        grid=(NP // bm,),
        in_specs=[pl.BlockSpec((bm, D), lambda i: (i, 0)),
                  pl.BlockSpec((D, D), lambda i: (0, 0)),
                  pl.BlockSpec((bm, D), lambda i: (i, 0))],
        out_specs=pl.BlockSpec((bm, D), lambda i: (i, 0)),
        out_shape=jax.ShapeDtypeStruct((NP, D), jnp.float32),
    )(x, w, d2)


def _layer2_body(a0_ref, a1_ref, h1p_ref, d2_ref, b_ref, w_ref, o_ref):
    z = (a0_ref[...] + a1_ref[...] + h1p_ref[...]) * d2_ref[...] + b_ref[0:1, :]
    z = jnp.maximum(z, 0.0)
    o_ref[...] = jnp.dot(z, w_ref[...],
                         preferred_element_type=jnp.float32) * d2_ref[...]


def _layer2_call(a0, a1, h1p, d2, b8, w):
    bm = 512
    return pl.pallas_call(
        _layer2_body,
        grid=(NP // bm,),
        in_specs=[pl.BlockSpec((bm, D), lambda i: (i, 0)),
                  pl.BlockSpec((bm, D), lambda i: (i, 0)),
                  pl.BlockSpec((bm, D), lambda i: (i, 0)),
                  pl.BlockSpec((bm, D), lambda i: (i, 0)),
                  pl.BlockSpec((8, D), lambda i: (0, 0)),
                  pl.BlockSpec((D, D), lambda i: (0, 0))],
        out_specs=pl.BlockSpec((bm, D), lambda i: (i, 0)),
        out_shape=jax.ShapeDtypeStruct((NP, D), jnp.float32),
    )(a0, a1, h1p, d2, b8, w)


@functools.partial(
    pl.kernel,
    out_type=(jax.ShapeDtypeStruct((NP, D), jnp.float32),
              jax.ShapeDtypeStruct((NP, D), jnp.float32)),
    mesh=_mesh,
    scratch_types=[
        pltpu.VMEM((G, CHUNK), jnp.int32),
        pltpu.VMEM((G, CHUNK), jnp.int32),
        pltpu.VMEM((G, CHUNK), jnp.float32),
        pltpu.VMEM((CHUNK, D), jnp.float32),
        pltpu.VMEM((CHUNK, D), jnp.float32),
        pltpu.VMEM_SHARED((NP, D), jnp.float32),
        pltpu.SemaphoreType.DMA,
        pltpu.SemaphoreType.DMA,
    ],
)
def _msg_kernel(h_hbm, src_hbm, dst_hbm, ew_hbm,
                agg0_hbm, agg1_hbm,
                src_s, dst_s, ew_s, rows0, rows1, acc, gsem, ssem):
    cid = lax.axis_index("c")
    tid = lax.axis_index("s")
    wid = _wid()

    def zrow(r, _):
        for j in range(D // L):
            rows0[r, pl.ds(j * L, L)] = jnp.zeros((L,), jnp.float32)
        return 0
    lax.fori_loop(0, CHUNK, zrow, 0)
    for i in range(ROWS_T // CHUNK):
        pltpu.sync_copy(rows0, acc.at[pl.ds(tid * ROWS_T + i * CHUNK, CHUNK)])
    plsc.subcore_barrier()

    bufs = (rows0, rows1)

    def group(g, _):
        pltpu.sync_copy(src_hbm.at[wid, pl.ds(g * G, G)], src_s)
        pltpu.sync_copy(dst_hbm.at[wid, pl.ds(g * G, G)], dst_s)
        pltpu.sync_copy(ew_hbm.at[wid, pl.ds(g * G, G)], ew_s)

        gd = {0: pltpu.async_copy(h_hbm.at[src_s.at[0]], bufs[0], gsem)}
        sd = {}
        for k2 in range(G):
            rb = bufs[k2 % 2]
            gd[k2].wait()
            if k2 + 1 < G:
                if k2 >= 1:
                    sd[k2 - 1].wait()
                gd[k2 + 1] = pltpu.async_copy(
                    h_hbm.at[src_s.at[k2 + 1]], bufs[(k2 + 1) % 2], gsem)

            def scale(jg, _):
                sv = ew_s[k2, pl.ds(jg * L, L)]
                for r2 in range(L):
                    sc = lax.broadcast(sv[r2], (L,))
                    r = jg * L + r2
                    for j in range(D // L):
                        sl = pl.ds(j * L, L)
                        rb[r, sl] = rb[r, sl] * sc
                return 0
            lax.fori_loop(0, CHUNK // L, scale, 0)
            sd[k2] = pltpu.async_copy(rb, acc.at[dst_s.at[k2]], ssem, add=True)
        sd[G - 2].wait()
        sd[G - 1].wait()
        return 0
    lax.fori_loop(0, NG, group, 0)
    plsc.subcore_barrier()

    @pl.when(cid == 0)
    def _():
        for i in range(ROWS_T // CHUNK):
            r0 = tid * ROWS_T + i * CHUNK
            pltpu.sync_copy(acc.at[pl.ds(r0, CHUNK)],
                            agg0_hbm.at[pl.ds(r0, CHUNK)])

    @pl.when(cid == 1)
    def _():
        for i in range(ROWS_T // CHUNK):
            r0 = tid * ROWS_T + i * CHUNK
            pltpu.sync_copy(acc.at[pl.ds(r0, CHUNK)],
                            agg1_hbm.at[pl.ds(r0, CHUNK)])


@functools.partial(
    pl.kernel,
    out_type=(jax.ShapeDtypeStruct((MASK_P, D), jnp.float32),
              jax.ShapeDtypeStruct((MASK_P,), jnp.int32)),
    mesh=_mesh,
    scratch_types=[
        pltpu.VMEM((MPW,), jnp.int32),
        pltpu.VMEM((MPW, D), jnp.float32),
        pltpu.VMEM((MPW, D), jnp.float32),
        pltpu.VMEM((MPW, D), jnp.float32),
        pltpu.VMEM((MPW, D), jnp.float32),
        pltpu.VMEM((MPW,), jnp.int32),
        pltpu.VMEM((D,), jnp.float32),
        pltpu.SemaphoreType.DMA,
    ],
)
def _final_kernel(a0_hbm, a1_hbm, h2p_hbm, d2_hbm, b2_hbm, mask_hbm, y_hbm,
                  out_hbm, ym_hbm,
                  m_s, r0, r1, r2v, r3, yb, b2_s, sem):
    wid = _wid()
    pltpu.sync_copy(mask_hbm.at[wid], m_s)
    pltpu.sync_copy(b2_hbm, b2_s)
    g1 = pltpu.async_copy(a0_hbm.at[m_s], r0, sem)
    g2 = pltpu.async_copy(a1_hbm.at[m_s], r1, sem)
    g3 = pltpu.async_copy(h2p_hbm.at[m_s], r2v, sem)
    g4 = pltpu.async_copy(d2_hbm.at[m_s], r3, sem)
    g5 = pltpu.async_copy(y_hbm.at[m_s], yb, sem)
    g1.wait()
    g2.wait()
    g3.wait()
    g4.wait()
    g5.wait()

    def body(r, _):
        for j in range(D // L):
            sl = pl.ds(j * L, L)
            r0[r, sl] = (r0[r, sl] + r1[r, sl] + r2v[r, sl]) * r3[r, sl] \
                + b2_s[sl]
        return 0
    lax.fori_loop(0, MPW, body, 0)
    pltpu.sync_copy(r0, out_hbm.at[pl.ds(wid * MPW, MPW)])
    pltpu.sync_copy(yb, ym_hbm.at[pl.ds(wid * MPW, MPW)])


def kernel(x, edge_index, edge_weight, mask_idx, y, W1, b1, W2, b2):
    src = edge_index[0].astype(jnp.int32)
    dst = edge_index[1].astype(jnp.int32)
    ew = edge_weight.astype(jnp.float32)

    srcp = jnp.pad(src, (0, EP - E_REAL)).reshape(NW, NCH, CHUNK)
    dstp = jnp.pad(dst, (0, EP - E_REAL)).reshape(NW, NCH, CHUNK)
    ewp = jnp.pad(ew, (0, EP - E_REAL)).reshape(NW, NCH, CHUNK)

    x_p = jnp.pad(x, ((0, NP - N_NODES), (0, 0)))
    mask_p = jnp.pad(mask_idx.astype(jnp.int32),
                     (0, MASK_P - mask_idx.shape[0])).reshape(NW, MPW)
    b1_8 = jnp.broadcast_to(b1.reshape(1, D), (8, D))

    degp0, degp1 = _deg_kernel(dstp, ewp)
    dinv2d = _dinv_call(degp0.reshape(NP // D, D), degp1.reshape(NP // D, D))
    d2 = jnp.broadcast_to(dinv2d.reshape(NP)[:, None], (NP, D))

    h1p = _mm_call(x_p, W1, d2)
    a0, a1 = _msg_kernel(h1p, srcp, dstp, ewp)
    h2p = _layer2_call(a0, a1, h1p, d2, b1_8, W2)
    c0, c1 = _msg_kernel(h2p, srcp, dstp, ewp)
    out_p, ym_p = _final_kernel(c0, c1, h2p, d2, b2, mask_p,
                                y.astype(jnp.int32))
    n_mask = mask_idx.shape[0]
    return out_p[:n_mask], ym_p[:n_mask]


# speedup vs baseline: 9.5486x; 1.1056x over previous
import functools

import jax
import jax.numpy as jnp
from jax import lax
from jax.experimental import pallas as pl
from jax.experimental.pallas import tpu as pltpu
from jax.experimental.pallas import tpu_sc as plsc

N_NODES = 10000
NP = 10240
D = 128
E_REAL = 320000
EP = 327680
NC = 2
NS = 16
NW = NC * NS
L = 16
CHUNK = 64
NCH = EP // NW // CHUNK
G = 16
NG = NCH // G
NBUF = 4
ROWS_T = NP // NS
MASK_P = 1024
MPW = MASK_P // NW

_mesh = plsc.VectorSubcoreMesh(core_axis_name="c", subcore_axis_name="s")


def _wid():
    return lax.axis_index("s") * NC + lax.axis_index("c")


@functools.partial(
    pl.kernel,
    out_type=(jax.ShapeDtypeStruct((NP,), jnp.float32),
              jax.ShapeDtypeStruct((NP,), jnp.float32)),
    mesh=_mesh,
    scratch_types=[
        pltpu.VMEM((NCH, CHUNK), jnp.int32),
        pltpu.VMEM((NCH, CHUNK), jnp.float32),
        pltpu.VMEM((ROWS_T,), jnp.float32),
        pltpu.VMEM_SHARED((NP,), jnp.float32),
    ],
)
def _deg_kernel(dst_hbm, ew_hbm, degp0_hbm, degp1_hbm, dst_s, ew_s, zb, dacc):
    cid = lax.axis_index("c")
    tid = lax.axis_index("s")
    wid = _wid()
    pltpu.sync_copy(dst_hbm.at[wid], dst_s)
    pltpu.sync_copy(ew_hbm.at[wid], ew_s)

    def zbody(r, _):
        zb[pl.ds(r * L, L)] = jnp.zeros((L,), jnp.float32)
        return 0
    lax.fori_loop(0, ROWS_T // L, zbody, 0)
    pltpu.sync_copy(zb, dacc.at[pl.ds(tid * ROWS_T, ROWS_T)])
    plsc.subcore_barrier()

    def body(k, _):
        pltpu.sync_copy(ew_s.at[k], dacc.at[dst_s.at[k]], add=True)
        return 0
    lax.fori_loop(0, NCH, body, 0)
    plsc.subcore_barrier()

    @pl.when(cid == 0)
    def _():
        pltpu.sync_copy(dacc.at[pl.ds(tid * ROWS_T, ROWS_T)],
                        degp0_hbm.at[pl.ds(tid * ROWS_T, ROWS_T)])

    @pl.when(cid == 1)
    def _():
        pltpu.sync_copy(dacc.at[pl.ds(tid * ROWS_T, ROWS_T)],
                        degp1_hbm.at[pl.ds(tid * ROWS_T, ROWS_T)])


def _dinv_body(d0_ref, d1_ref, o_ref):
    deg = d0_ref[...] + d1_ref[...] + 1.0
    o_ref[...] = lax.rsqrt(deg)


def _dinv_call(d0, d1):
    return pl.pallas_call(
        _dinv_body,
        out_shape=jax.ShapeDtypeStruct((NP // D, D), jnp.float32),
    )(d0, d1)


def _mm_body(x_ref, w_ref, d2_ref, o_ref):
    o_ref[...] = jnp.dot(x_ref[...], w_ref[...],
                         preferred_element_type=jnp.float32) * d2_ref[...]


def _mm_call(x, w, d2):
    bm = 512
    return pl.pallas_call(
        _mm_body,
        grid=(NP // bm,),
        in_specs=[pl.BlockSpec((bm, D), lambda i: (i, 0)),
                  pl.BlockSpec((D, D), lambda i: (0, 0)),
                  pl.BlockSpec((bm, D), lambda i: (i, 0))],
        out_specs=pl.BlockSpec((bm, D), lambda i: (i, 0)),
        out_shape=jax.ShapeDtypeStruct((NP, D), jnp.float32),
    )(x, w, d2)


def _layer2_body(a0_ref, a1_ref, h1p_ref, d2_ref, b_ref, w_ref, o_ref):
    z = (a0_ref[...] + a1_ref[...] + h1p_ref[...]) * d2_ref[...] + b_ref[0:1, :]
    z = jnp.maximum(z, 0.0)
    o_ref[...] = jnp.dot(z, w_ref[...],
                         preferred_element_type=jnp.float32) * d2_ref[...]


def _layer2_call(a0, a1, h1p, d2, b8, w):
    bm = 512
    return pl.pallas_call(
        _layer2_body,
        grid=(NP // bm,),
        in_specs=[pl.BlockSpec((bm, D), lambda i: (i, 0)),
                  pl.BlockSpec((bm, D), lambda i: (i, 0)),
                  pl.BlockSpec((bm, D), lambda i: (i, 0)),
                  pl.BlockSpec((bm, D), lambda i: (i, 0)),
                  pl.BlockSpec((8, D), lambda i: (0, 0)),
                  pl.BlockSpec((D, D), lambda i: (0, 0))],
        out_specs=pl.BlockSpec((bm, D), lambda i: (i, 0)),
        out_shape=jax.ShapeDtypeStruct((NP, D), jnp.float32),
    )(a0, a1, h1p, d2, b8, w)


@functools.partial(
    pl.kernel,
    out_type=(jax.ShapeDtypeStruct((NP, D), jnp.float32),
              jax.ShapeDtypeStruct((NP, D), jnp.float32)),
    mesh=_mesh,
    scratch_types=[
        pltpu.VMEM((G, CHUNK), jnp.int32),
        pltpu.VMEM((G, CHUNK), jnp.int32),
        pltpu.VMEM((G, CHUNK), jnp.float32),
        pltpu.VMEM((CHUNK, D), jnp.float32),
        pltpu.VMEM((CHUNK, D), jnp.float32),
        pltpu.VMEM((CHUNK, D), jnp.float32),
        pltpu.VMEM((CHUNK, D), jnp.float32),
        pltpu.VMEM_SHARED((NP, D), jnp.float32),
        pltpu.SemaphoreType.DMA,
        pltpu.SemaphoreType.DMA,
    ],
)
def _msg_kernel(h_hbm, src_hbm, dst_hbm, ew_hbm,
                agg0_hbm, agg1_hbm,
                src_s, dst_s, ew_s, rows0, rows1, rows2, rows3,
                acc, gsem, ssem):
    cid = lax.axis_index("c")
    tid = lax.axis_index("s")
    wid = _wid()

    def zrow(r, _):
        for j in range(D // L):
            rows0[r, pl.ds(j * L, L)] = jnp.zeros((L,), jnp.float32)
        return 0
    lax.fori_loop(0, CHUNK, zrow, 0)
    for i in range(ROWS_T // CHUNK):
        pltpu.sync_copy(rows0, acc.at[pl.ds(tid * ROWS_T + i * CHUNK, CHUNK)])
    plsc.subcore_barrier()

    bufs = (rows0, rows1, rows2, rows3)
    AHEAD = NBUF - 1

    def group(g, _):
        pltpu.sync_copy(src_hbm.at[wid, pl.ds(g * G, G)], src_s)
        pltpu.sync_copy(dst_hbm.at[wid, pl.ds(g * G, G)], dst_s)
        pltpu.sync_copy(ew_hbm.at[wid, pl.ds(g * G, G)], ew_s)

        gd = {}
        sd = {}
        for k2 in range(AHEAD):
            gd[k2] = pltpu.async_copy(
                h_hbm.at[src_s.at[k2]], bufs[k2 % NBUF], gsem)
        for k2 in range(G):
            rb = bufs[k2 % NBUF]
            gd.pop(k2).wait()
            if k2 + AHEAD < G:
                if (k2 - 1) in sd:
                    sd.pop(k2 - 1).wait()
                gd[k2 + AHEAD] = pltpu.async_copy(
                    h_hbm.at[src_s.at[k2 + AHEAD]],
                    bufs[(k2 + AHEAD) % NBUF], gsem)

            def scale(jg, _):
                sv = ew_s[k2, pl.ds(jg * L, L)]
                for r2 in range(L):
                    sc = lax.broadcast(sv[r2], (L,))
                    r = jg * L + r2
                    for j in range(D // L):
                        sl = pl.ds(j * L, L)
                        rb[r, sl] = rb[r, sl] * sc
                return 0
            lax.fori_loop(0, CHUNK // L, scale, 0)
            sd[k2] = pltpu.async_copy(rb, acc.at[dst_s.at[k2]], ssem,
                                      add=True)
        for d in sd.values():
            d.wait()
        return 0
    lax.fori_loop(0, NG, group, 0)
    plsc.subcore_barrier()

    @pl.when(cid == 0)
    def _():
        for i in range(ROWS_T // CHUNK):
            r0 = tid * ROWS_T + i * CHUNK
            pltpu.sync_copy(acc.at[pl.ds(r0, CHUNK)],
                            agg0_hbm.at[pl.ds(r0, CHUNK)])

    @pl.when(cid == 1)
    def _():
        for i in range(ROWS_T // CHUNK):
            r0 = tid * ROWS_T + i * CHUNK
            pltpu.sync_copy(acc.at[pl.ds(r0, CHUNK)],
                            agg1_hbm.at[pl.ds(r0, CHUNK)])


@functools.partial(
    pl.kernel,
    out_type=(jax.ShapeDtypeStruct((MASK_P, D), jnp.float32),
              jax.ShapeDtypeStruct((MASK_P,), jnp.int32)),
    mesh=_mesh,
    scratch_types=[
        pltpu.VMEM((MPW,), jnp.int32),
        pltpu.VMEM((MPW, D), jnp.float32),
        pltpu.VMEM((MPW, D), jnp.float32),
        pltpu.VMEM((MPW, D), jnp.float32),
        pltpu.VMEM((MPW, D), jnp.float32),
        pltpu.VMEM((MPW,), jnp.int32),
        pltpu.VMEM((D,), jnp.float32),
        pltpu.SemaphoreType.DMA,
    ],
)
def _final_kernel(a0_hbm, a1_hbm, h2p_hbm, d2_hbm, b2_hbm, mask_hbm, y_hbm,
                  out_hbm, ym_hbm,
                  m_s, r0, r1, r2v, r3, yb, b2_s, sem):
    wid = _wid()
    pltpu.sync_copy(mask_hbm.at[wid], m_s)
    pltpu.sync_copy(b2_hbm, b2_s)
    g1 = pltpu.async_copy(a0_hbm.at[m_s], r0, sem)
    g2 = pltpu.async_copy(a1_hbm.at[m_s], r1, sem)
    g3 = pltpu.async_copy(h2p_hbm.at[m_s], r2v, sem)
    g4 = pltpu.async_copy(d2_hbm.at[m_s], r3, sem)
    g5 = pltpu.async_copy(y_hbm.at[m_s], yb, sem)
    g1.wait()
    g2.wait()
    g3.wait()
    g4.wait()
    g5.wait()

    def body(r, _):
        for j in range(D // L):
            sl = pl.ds(j * L, L)
            r0[r, sl] = (r0[r, sl] + r1[r, sl] + r2v[r, sl]) * r3[r, sl] \
                + b2_s[sl]
        return 0
    lax.fori_loop(0, MPW, body, 0)
    pltpu.sync_copy(r0, out_hbm.at[pl.ds(wid * MPW, MPW)])
    pltpu.sync_copy(yb, ym_hbm.at[pl.ds(wid * MPW, MPW)])


def kernel(x, edge_index, edge_weight, mask_idx, y, W1, b1, W2, b2):
    src = edge_index[0].astype(jnp.int32)
    dst = edge_index[1].astype(jnp.int32)
    ew = edge_weight.astype(jnp.float32)

    srcp = jnp.pad(src, (0, EP - E_REAL)).reshape(NW, NCH, CHUNK)
    dstp = jnp.pad(dst, (0, EP - E_REAL)).reshape(NW, NCH, CHUNK)
    ewp = jnp.pad(ew, (0, EP - E_REAL)).reshape(NW, NCH, CHUNK)

    x_p = jnp.pad(x, ((0, NP - N_NODES), (0, 0)))
    mask_p = jnp.pad(mask_idx.astype(jnp.int32),
                     (0, MASK_P - mask_idx.shape[0])).reshape(NW, MPW)
    b1_8 = jnp.broadcast_to(b1.reshape(1, D), (8, D))

    degp0, degp1 = _deg_kernel(dstp, ewp)
    dinv2d = _dinv_call(degp0.reshape(NP // D, D), degp1.reshape(NP // D, D))
    d2 = jnp.broadcast_to(dinv2d.reshape(NP)[:, None], (NP, D))

    h1p = _mm_call(x_p, W1, d2)
    a0, a1 = _msg_kernel(h1p, srcp, dstp, ewp)
    h2p = _layer2_call(a0, a1, h1p, d2, b1_8, W2)
    c0, c1 = _msg_kernel(h2p, srcp, dstp, ewp)
    out_p, ym_p = _final_kernel(c0, c1, h2p, d2, b2, mask_p,
                                y.astype(jnp.int32))
    n_mask = mask_idx.shape[0]
    return out_p[:n_mask], ym_p[:n_mask]


# speedup vs baseline: 9.5655x; 1.0018x over previous
import functools

import jax
import jax.numpy as jnp
from jax import lax
from jax.experimental import pallas as pl
from jax.experimental.pallas import tpu as pltpu
from jax.experimental.pallas import tpu_sc as plsc

N_NODES = 10000
NP = 10240
D = 128
E_REAL = 320000
EP = 327680
NC = 2
NS = 16
NW = NC * NS
L = 16
CHUNK = 64
NCH = EP // NW // CHUNK
G = 16
NG = NCH // G
NBUF = 5
ROWS_T = NP // NS
MASK_P = 1024
MPW = MASK_P // NW

_mesh = plsc.VectorSubcoreMesh(core_axis_name="c", subcore_axis_name="s")


def _wid():
    return lax.axis_index("s") * NC + lax.axis_index("c")


@functools.partial(
    pl.kernel,
    out_type=(jax.ShapeDtypeStruct((NP,), jnp.float32),
              jax.ShapeDtypeStruct((NP,), jnp.float32)),
    mesh=_mesh,
    scratch_types=[
        pltpu.VMEM((NCH, CHUNK), jnp.int32),
        pltpu.VMEM((NCH, CHUNK), jnp.float32),
        pltpu.VMEM((ROWS_T,), jnp.float32),
        pltpu.VMEM_SHARED((NP,), jnp.float32),
    ],
)
def _deg_kernel(dst_hbm, ew_hbm, degp0_hbm, degp1_hbm, dst_s, ew_s, zb, dacc):
    cid = lax.axis_index("c")
    tid = lax.axis_index("s")
    wid = _wid()
    pltpu.sync_copy(dst_hbm.at[wid], dst_s)
    pltpu.sync_copy(ew_hbm.at[wid], ew_s)

    def zbody(r, _):
        zb[pl.ds(r * L, L)] = jnp.zeros((L,), jnp.float32)
        return 0
    lax.fori_loop(0, ROWS_T // L, zbody, 0)
    pltpu.sync_copy(zb, dacc.at[pl.ds(tid * ROWS_T, ROWS_T)])
    plsc.subcore_barrier()

    def body(k, _):
        pltpu.sync_copy(ew_s.at[k], dacc.at[dst_s.at[k]], add=True)
        return 0
    lax.fori_loop(0, NCH, body, 0)
    plsc.subcore_barrier()

    @pl.when(cid == 0)
    def _():
        pltpu.sync_copy(dacc.at[pl.ds(tid * ROWS_T, ROWS_T)],
                        degp0_hbm.at[pl.ds(tid * ROWS_T, ROWS_T)])

    @pl.when(cid == 1)
    def _():
        pltpu.sync_copy(dacc.at[pl.ds(tid * ROWS_T, ROWS_T)],
                        degp1_hbm.at[pl.ds(tid * ROWS_T, ROWS_T)])


def _dinv_body(d0_ref, d1_ref, o_ref):
    deg = d0_ref[...] + d1_ref[...] + 1.0
    o_ref[...] = lax.rsqrt(deg)


def _dinv_call(d0, d1):
    return pl.pallas_call(
        _dinv_body,
        out_shape=jax.ShapeDtypeStruct((NP // D, D), jnp.float32),
    )(d0, d1)


def _mm_body(x_ref, w_ref, d2_ref, o_ref):
    o_ref[...] = jnp.dot(x_ref[...], w_ref[...],
                         preferred_element_type=jnp.float32) * d2_ref[...]


def _mm_call(x, w, d2):
    bm = 512
    return pl.pallas_call(
        _mm_body,
        grid=(NP // bm,),
        in_specs=[pl.BlockSpec((bm, D), lambda i: (i, 0)),
                  pl.BlockSpec((D, D), lambda i: (0, 0)),
                  pl.BlockSpec((bm, D), lambda i: (i, 0))],
        out_specs=pl.BlockSpec((bm, D), lambda i: (i, 0)),
        out_shape=jax.ShapeDtypeStruct((NP, D), jnp.float32),
    )(x, w, d2)


def _layer2_body(a0_ref, a1_ref, h1p_ref, d2_ref, b_ref, w_ref, o_ref):
    z = (a0_ref[...] + a1_ref[...] + h1p_ref[...]) * d2_ref[...] + b_ref[0:1, :]
    z = jnp.maximum(z, 0.0)
    o_ref[...] = jnp.dot(z, w_ref[...],
                         preferred_element_type=jnp.float32) * d2_ref[...]


def _layer2_call(a0, a1, h1p, d2, b8, w):
    bm = 512
    return pl.pallas_call(
        _layer2_body,
        grid=(NP // bm,),
        in_specs=[pl.BlockSpec((bm, D), lambda i: (i, 0)),
                  pl.BlockSpec((bm, D), lambda i: (i, 0)),
                  pl.BlockSpec((bm, D), lambda i: (i, 0)),
                  pl.BlockSpec((bm, D), lambda i: (i, 0)),
                  pl.BlockSpec((8, D), lambda i: (0, 0)),
                  pl.BlockSpec((D, D), lambda i: (0, 0))],
        out_specs=pl.BlockSpec((bm, D), lambda i: (i, 0)),
        out_shape=jax.ShapeDtypeStruct((NP, D), jnp.float32),
    )(a0, a1, h1p, d2, b8, w)


@functools.partial(
    pl.kernel,
    out_type=(jax.ShapeDtypeStruct((NP, D), jnp.float32),
              jax.ShapeDtypeStruct((NP, D), jnp.float32)),
    mesh=_mesh,
    scratch_types=[
        pltpu.VMEM((G, CHUNK), jnp.int32),
        pltpu.VMEM((G, CHUNK), jnp.int32),
        pltpu.VMEM((G, CHUNK), jnp.float32),
        pltpu.VMEM((CHUNK, D), jnp.float32),
        pltpu.VMEM((CHUNK, D), jnp.float32),
        pltpu.VMEM((CHUNK, D), jnp.float32),
        pltpu.VMEM((CHUNK, D), jnp.float32),
        pltpu.VMEM((CHUNK, D), jnp.float32),
        pltpu.VMEM_SHARED((NP, D), jnp.float32),
        pltpu.SemaphoreType.DMA,
        pltpu.SemaphoreType.DMA,
    ],
)
def _msg_kernel(h_hbm, src_hbm, dst_hbm, ew_hbm,
                agg0_hbm, agg1_hbm,
                src_s, dst_s, ew_s, rows0, rows1, rows2, rows3, rows4,
                acc, gsem, ssem):
    cid = lax.axis_index("c")
    tid = lax.axis_index("s")
    wid = _wid()

    def zrow(r, _):
        for j in range(D // L):
            rows0[r, pl.ds(j * L, L)] = jnp.zeros((L,), jnp.float32)
        return 0
    lax.fori_loop(0, CHUNK, zrow, 0)
    for i in range(ROWS_T // CHUNK):
        pltpu.sync_copy(rows0, acc.at[pl.ds(tid * ROWS_T + i * CHUNK, CHUNK)])
    plsc.subcore_barrier()

    bufs = (rows0, rows1, rows2, rows3, rows4)
    AHEAD = NBUF - 1

    def group(g, _):
        pltpu.sync_copy(src_hbm.at[wid, pl.ds(g * G, G)], src_s)
        pltpu.sync_copy(dst_hbm.at[wid, pl.ds(g * G, G)], dst_s)
        pltpu.sync_copy(ew_hbm.at[wid, pl.ds(g * G, G)], ew_s)

        gd = {}
        sd = {}
        for k2 in range(AHEAD):
            gd[k2] = pltpu.async_copy(
                h_hbm.at[src_s.at[k2]], bufs[k2 % NBUF], gsem)
        for k2 in range(G):
            rb = bufs[k2 % NBUF]
            gd.pop(k2).wait()
            if k2 + AHEAD < G:
                if (k2 - 1) in sd:
                    sd.pop(k2 - 1).wait()
                gd[k2 + AHEAD] = pltpu.async_copy(
                    h_hbm.at[src_s.at[k2 + AHEAD]],
                    bufs[(k2 + AHEAD) % NBUF], gsem)

            def scale(jg, _):
                sv = ew_s[k2, pl.ds(jg * L, L)]
                for r2 in range(L):
                    sc = lax.broadcast(sv[r2], (L,))
                    r = jg * L + r2
                    for j in range(D // L):
                        sl = pl.ds(j * L, L)
                        rb[r, sl] = rb[r, sl] * sc
                return 0
            lax.fori_loop(0, CHUNK // L, scale, 0)
            sd[k2] = pltpu.async_copy(rb, acc.at[dst_s.at[k2]], ssem,
                                      add=True)
        for d in sd.values():
            d.wait()
        return 0
    lax.fori_loop(0, NG, group, 0)
    plsc.subcore_barrier()

    @pl.when(cid == 0)
    def _():
        for i in range(ROWS_T // CHUNK):
            r0 = tid * ROWS_T + i * CHUNK
            pltpu.sync_copy(acc.at[pl.ds(r0, CHUNK)],
                            agg0_hbm.at[pl.ds(r0, CHUNK)])

    @pl.when(cid == 1)
    def _():
        for i in range(ROWS_T // CHUNK):
            r0 = tid * ROWS_T + i * CHUNK
            pltpu.sync_copy(acc.at[pl.ds(r0, CHUNK)],
                            agg1_hbm.at[pl.ds(r0, CHUNK)])


@functools.partial(
    pl.kernel,
    out_type=(jax.ShapeDtypeStruct((MASK_P, D), jnp.float32),
              jax.ShapeDtypeStruct((MASK_P,), jnp.int32)),
    mesh=_mesh,
    scratch_types=[
        pltpu.VMEM((MPW,), jnp.int32),
        pltpu.VMEM((MPW, D), jnp.float32),
        pltpu.VMEM((MPW, D), jnp.float32),
        pltpu.VMEM((MPW, D), jnp.float32),
        pltpu.VMEM((MPW, D), jnp.float32),
        pltpu.VMEM((MPW,), jnp.int32),
        pltpu.VMEM((D,), jnp.float32),
        pltpu.SemaphoreType.DMA,
    ],
)
def _final_kernel(a0_hbm, a1_hbm, h2p_hbm, d2_hbm, b2_hbm, mask_hbm, y_hbm,
                  out_hbm, ym_hbm,
                  m_s, r0, r1, r2v, r3, yb, b2_s, sem):
    wid = _wid()
    pltpu.sync_copy(mask_hbm.at[wid], m_s)
    pltpu.sync_copy(b2_hbm, b2_s)
    g1 = pltpu.async_copy(a0_hbm.at[m_s], r0, sem)
    g2 = pltpu.async_copy(a1_hbm.at[m_s], r1, sem)
    g3 = pltpu.async_copy(h2p_hbm.at[m_s], r2v, sem)
    g4 = pltpu.async_copy(d2_hbm.at[m_s], r3, sem)
    g5 = pltpu.async_copy(y_hbm.at[m_s], yb, sem)
    g1.wait()
    g2.wait()
    g3.wait()
    g4.wait()
    g5.wait()

    def body(r, _):
        for j in range(D // L):
            sl = pl.ds(j * L, L)
            r0[r, sl] = (r0[r, sl] + r1[r, sl] + r2v[r, sl]) * r3[r, sl] \
                + b2_s[sl]
        return 0
    lax.fori_loop(0, MPW, body, 0)
    pltpu.sync_copy(r0, out_hbm.at[pl.ds(wid * MPW, MPW)])
    pltpu.sync_copy(yb, ym_hbm.at[pl.ds(wid * MPW, MPW)])


def kernel(x, edge_index, edge_weight, mask_idx, y, W1, b1, W2, b2):
    src = edge_index[0].astype(jnp.int32)
    dst = edge_index[1].astype(jnp.int32)
    ew = edge_weight.astype(jnp.float32)

    srcp = jnp.pad(src, (0, EP - E_REAL)).reshape(NW, NCH, CHUNK)
    dstp = jnp.pad(dst, (0, EP - E_REAL)).reshape(NW, NCH, CHUNK)
    ewp = jnp.pad(ew, (0, EP - E_REAL)).reshape(NW, NCH, CHUNK)

    x_p = jnp.pad(x, ((0, NP - N_NODES), (0, 0)))
    mask_p = jnp.pad(mask_idx.astype(jnp.int32),
                     (0, MASK_P - mask_idx.shape[0])).reshape(NW, MPW)
    b1_8 = jnp.broadcast_to(b1.reshape(1, D), (8, D))

    degp0, degp1 = _deg_kernel(dstp, ewp)
    dinv2d = _dinv_call(degp0.reshape(NP // D, D), degp1.reshape(NP // D, D))
    d2 = jnp.broadcast_to(dinv2d.reshape(NP)[:, None], (NP, D))

    h1p = _mm_call(x_p, W1, d2)
    a0, a1 = _msg_kernel(h1p, srcp, dstp, ewp)
    h2p = _layer2_call(a0, a1, h1p, d2, b1_8, W2)
    c0, c1 = _msg_kernel(h2p, srcp, dstp, ewp)
    out_p, ym_p = _final_kernel(c0, c1, h2p, d2, b2, mask_p,
                                y.astype(jnp.int32))
    n_mask = mask_idx.shape[0]
    return out_p[:n_mask], ym_p[:n_mask]


# speedup vs baseline: 9.7153x; 1.0157x over previous
import functools

import jax
import jax.numpy as jnp
from jax import lax
from jax.experimental import pallas as pl
from jax.experimental.pallas import tpu as pltpu
from jax.experimental.pallas import tpu_sc as plsc

N_NODES = 10000
NP = 10240
D = 128
E_REAL = 320000
EP = 327680
NC = 2
NS = 16
NW = NC * NS
L = 16
CHUNK = 64
NCH = EP // NW // CHUNK
G = 16
NG = NCH // G
NBUF = 5
ROWS_T = NP // NS
MASK_P = 1024
MPW = MASK_P // NW

_mesh = plsc.VectorSubcoreMesh(core_axis_name="c", subcore_axis_name="s")


def _wid():
    return lax.axis_index("s") * NC + lax.axis_index("c")


@functools.partial(
    pl.kernel,
    out_type=(jax.ShapeDtypeStruct((NP,), jnp.float32),
              jax.ShapeDtypeStruct((NP,), jnp.float32)),
    mesh=_mesh,
    scratch_types=[
        pltpu.VMEM((NCH, CHUNK), jnp.int32),
        pltpu.VMEM((NCH, CHUNK), jnp.float32),
        pltpu.VMEM((ROWS_T,), jnp.float32),
        pltpu.VMEM_SHARED((NP,), jnp.float32),
    ],
)
def _deg_kernel(dst_hbm, ew_hbm, degp0_hbm, degp1_hbm, dst_s, ew_s, zb, dacc):
    cid = lax.axis_index("c")
    tid = lax.axis_index("s")
    wid = _wid()
    pltpu.sync_copy(dst_hbm.at[wid], dst_s)
    pltpu.sync_copy(ew_hbm.at[wid], ew_s)

    def zbody(r, _):
        zb[pl.ds(r * L, L)] = jnp.zeros((L,), jnp.float32)
        return 0
    lax.fori_loop(0, ROWS_T // L, zbody, 0)
    pltpu.sync_copy(zb, dacc.at[pl.ds(tid * ROWS_T, ROWS_T)])
    plsc.subcore_barrier()

    def body(k, _):
        pltpu.sync_copy(ew_s.at[k], dacc.at[dst_s.at[k]], add=True)
        return 0
    lax.fori_loop(0, NCH, body, 0)
    plsc.subcore_barrier()

    @pl.when(cid == 0)
    def _():
        pltpu.sync_copy(dacc.at[pl.ds(tid * ROWS_T, ROWS_T)],
                        degp0_hbm.at[pl.ds(tid * ROWS_T, ROWS_T)])

    @pl.when(cid == 1)
    def _():
        pltpu.sync_copy(dacc.at[pl.ds(tid * ROWS_T, ROWS_T)],
                        degp1_hbm.at[pl.ds(tid * ROWS_T, ROWS_T)])


def _dinv_body(d0_ref, d1_ref, o_ref):
    deg = d0_ref[...] + d1_ref[...] + 1.0
    o_ref[...] = lax.rsqrt(deg)


def _dinv_call(d0, d1):
    return pl.pallas_call(
        _dinv_body,
        out_shape=jax.ShapeDtypeStruct((NP // D, D), jnp.float32),
    )(d0, d1)


def _mm_body(x_ref, w_ref, d2_ref, o_ref):
    o_ref[...] = jnp.dot(x_ref[...], w_ref[...],
                         preferred_element_type=jnp.float32) * d2_ref[...]


def _mm_call(x, w, d2):
    bm = 512
    return pl.pallas_call(
        _mm_body,
        grid=(NP // bm,),
        in_specs=[pl.BlockSpec((bm, D), lambda i: (i, 0)),
                  pl.BlockSpec((D, D), lambda i: (0, 0)),
                  pl.BlockSpec((bm, D), lambda i: (i, 0))],
        out_specs=pl.BlockSpec((bm, D), lambda i: (i, 0)),
        out_shape=jax.ShapeDtypeStruct((NP, D), jnp.float32),
    )(x, w, d2)


def _layer2_body(a0_ref, a1_ref, h1p_ref, d2_ref, b_ref, w_ref, o_ref):
    z = (a0_ref[...] + a1_ref[...] + h1p_ref[...]) * d2_ref[...] + b_ref[0:1, :]
    z = jnp.maximum(z, 0.0)
    o_ref[...] = jnp.dot(z, w_ref[...],
                         preferred_element_type=jnp.float32) * d2_ref[...]


def _layer2_call(a0, a1, h1p, d2, b8, w):
    bm = 512
    return pl.pallas_call(
        _layer2_body,
        grid=(NP // bm,),
        in_specs=[pl.BlockSpec((bm, D), lambda i: (i, 0)),
                  pl.BlockSpec((bm, D), lambda i: (i, 0)),
                  pl.BlockSpec((bm, D), lambda i: (i, 0)),
                  pl.BlockSpec((bm, D), lambda i: (i, 0)),
                  pl.BlockSpec((8, D), lambda i: (0, 0)),
                  pl.BlockSpec((D, D), lambda i: (0, 0))],
        out_specs=pl.BlockSpec((bm, D), lambda i: (i, 0)),
        out_shape=jax.ShapeDtypeStruct((NP, D), jnp.float32),
    )(a0, a1, h1p, d2, b8, w)


@functools.partial(
    pl.kernel,
    out_type=(jax.ShapeDtypeStruct((NP, D), jnp.float32),
              jax.ShapeDtypeStruct((NP, D), jnp.float32)),
    mesh=_mesh,
    scratch_types=[
        pltpu.VMEM((G, CHUNK), jnp.int32),
        pltpu.VMEM((G, CHUNK), jnp.int32),
        pltpu.VMEM((G, CHUNK), jnp.float32),
        pltpu.VMEM((CHUNK, D), jnp.float32),
        pltpu.VMEM((CHUNK, D), jnp.float32),
        pltpu.VMEM((CHUNK, D), jnp.float32),
        pltpu.VMEM((CHUNK, D), jnp.float32),
        pltpu.VMEM((CHUNK, D), jnp.float32),
        pltpu.VMEM_SHARED((NP, D), jnp.float32),
        pltpu.SemaphoreType.DMA,
        pltpu.SemaphoreType.DMA,
    ],
)
def _msg_kernel(h_hbm, src_hbm, dst_hbm, ew_hbm,
                agg0_hbm, agg1_hbm,
                src_s, dst_s, ew_s, rows0, rows1, rows2, rows3, rows4,
                acc, gsem, ssem):
    cid = lax.axis_index("c")
    tid = lax.axis_index("s")
    wid = _wid()

    def zrow(r, _):
        for j in range(D // L):
            rows0[r, pl.ds(j * L, L)] = jnp.zeros((L,), jnp.float32)
        return 0
    lax.fori_loop(0, CHUNK, zrow, 0)
    for i in range(ROWS_T // CHUNK):
        pltpu.sync_copy(rows0, acc.at[pl.ds(tid * ROWS_T + i * CHUNK, CHUNK)])
    plsc.subcore_barrier()

    bufs = (rows0, rows1, rows2, rows3, rows4)
    AHEAD = NBUF - 1

    def group(g, _):
        i1 = pltpu.async_copy(src_hbm.at[wid, pl.ds(g * G, G)], src_s, gsem)
        i2 = pltpu.async_copy(dst_hbm.at[wid, pl.ds(g * G, G)], dst_s, gsem)
        i3 = pltpu.async_copy(ew_hbm.at[wid, pl.ds(g * G, G)], ew_s, gsem)
        i1.wait()
        i2.wait()
        i3.wait()

        gd = {}
        sd = {}
        for k2 in range(AHEAD):
            gd[k2] = pltpu.async_copy(
                h_hbm.at[src_s.at[k2]], bufs[k2 % NBUF], gsem)
        for k2 in range(G):
            rb = bufs[k2 % NBUF]
            gd.pop(k2).wait()
            if k2 + AHEAD < G:
                if (k2 - 1) in sd:
                    sd.pop(k2 - 1).wait()
                gd[k2 + AHEAD] = pltpu.async_copy(
                    h_hbm.at[src_s.at[k2 + AHEAD]],
                    bufs[(k2 + AHEAD) % NBUF], gsem)

            def scale(jg, _):
                sv = ew_s[k2, pl.ds(jg * L, L)]
                for r2 in range(L):
                    sc = lax.broadcast(sv[r2], (L,))
                    r = jg * L + r2
                    for j in range(D // L):
                        sl = pl.ds(j * L, L)
                        rb[r, sl] = rb[r, sl] * sc
                return 0
            lax.fori_loop(0, CHUNK // L, scale, 0)
            sd[k2] = pltpu.async_copy(rb, acc.at[dst_s.at[k2]], ssem,
                                      add=True)
        for d in sd.values():
            d.wait()
        return 0
    lax.fori_loop(0, NG, group, 0)
    plsc.subcore_barrier()

    @pl.when(cid == 0)
    def _():
        for i in range(ROWS_T // CHUNK):
            r0 = tid * ROWS_T + i * CHUNK
            pltpu.sync_copy(acc.at[pl.ds(r0, CHUNK)],
                            agg0_hbm.at[pl.ds(r0, CHUNK)])

    @pl.when(cid == 1)
    def _():
        for i in range(ROWS_T // CHUNK):
            r0 = tid * ROWS_T + i * CHUNK
            pltpu.sync_copy(acc.at[pl.ds(r0, CHUNK)],
                            agg1_hbm.at[pl.ds(r0, CHUNK)])


@functools.partial(
    pl.kernel,
    out_type=(jax.ShapeDtypeStruct((MASK_P, D), jnp.float32),
              jax.ShapeDtypeStruct((MASK_P,), jnp.int32)),
    mesh=_mesh,
    scratch_types=[
        pltpu.VMEM((MPW,), jnp.int32),
        pltpu.VMEM((MPW, D), jnp.float32),
        pltpu.VMEM((MPW, D), jnp.float32),
        pltpu.VMEM((MPW, D), jnp.float32),
        pltpu.VMEM((MPW, D), jnp.float32),
        pltpu.VMEM((MPW,), jnp.int32),
        pltpu.VMEM((D,), jnp.float32),
        pltpu.SemaphoreType.DMA,
    ],
)
def _final_kernel(a0_hbm, a1_hbm, h2p_hbm, d2_hbm, b2_hbm, mask_hbm, y_hbm,
                  out_hbm, ym_hbm,
                  m_s, r0, r1, r2v, r3, yb, b2_s, sem):
    wid = _wid()
    pltpu.sync_copy(mask_hbm.at[wid], m_s)
    pltpu.sync_copy(b2_hbm, b2_s)
    g1 = pltpu.async_copy(a0_hbm.at[m_s], r0, sem)
    g2 = pltpu.async_copy(a1_hbm.at[m_s], r1, sem)
    g3 = pltpu.async_copy(h2p_hbm.at[m_s], r2v, sem)
    g4 = pltpu.async_copy(d2_hbm.at[m_s], r3, sem)
    g5 = pltpu.async_copy(y_hbm.at[m_s], yb, sem)
    g1.wait()
    g2.wait()
    g3.wait()
    g4.wait()
    g5.wait()

    def body(r, _):
        for j in range(D // L):
            sl = pl.ds(j * L, L)
            r0[r, sl] = (r0[r, sl] + r1[r, sl] + r2v[r, sl]) * r3[r, sl] \
                + b2_s[sl]
        return 0
    lax.fori_loop(0, MPW, body, 0)
    pltpu.sync_copy(r0, out_hbm.at[pl.ds(wid * MPW, MPW)])
    pltpu.sync_copy(yb, ym_hbm.at[pl.ds(wid * MPW, MPW)])


def kernel(x, edge_index, edge_weight, mask_idx, y, W1, b1, W2, b2):
    src = edge_index[0].astype(jnp.int32)
    dst = edge_index[1].astype(jnp.int32)
    ew = edge_weight.astype(jnp.float32)

    srcp = jnp.pad(src, (0, EP - E_REAL)).reshape(NW, NCH, CHUNK)
    dstp = jnp.pad(dst, (0, EP - E_REAL)).reshape(NW, NCH, CHUNK)
    ewp = jnp.pad(ew, (0, EP - E_REAL)).reshape(NW, NCH, CHUNK)

    x_p = jnp.pad(x, ((0, NP - N_NODES), (0, 0)))
    mask_p = jnp.pad(mask_idx.astype(jnp.int32),
                     (0, MASK_P - mask_idx.shape[0])).reshape(NW, MPW)
    b1_8 = jnp.broadcast_to(b1.reshape(1, D), (8, D))

    degp0, degp1 = _deg_kernel(dstp, ewp)
    dinv2d = _dinv_call(degp0.reshape(NP // D, D), degp1.reshape(NP // D, D))
    d2 = jnp.broadcast_to(dinv2d.reshape(NP)[:, None], (NP, D))

    h1p = _mm_call(x_p, W1, d2)
    a0, a1 = _msg_kernel(h1p, srcp, dstp, ewp)
    h2p = _layer2_call(a0, a1, h1p, d2, b1_8, W2)
    c0, c1 = _msg_kernel(h2p, srcp, dstp, ewp)
    out_p, ym_p = _final_kernel(c0, c1, h2p, d2, b2, mask_p,
                                y.astype(jnp.int32))
    n_mask = mask_idx.shape[0]
    return out_p[:n_mask], ym_p[:n_mask]


# speedup vs baseline: 9.7438x; 1.0029x over previous
import functools

import jax
import jax.numpy as jnp
from jax import lax
from jax.experimental import pallas as pl
from jax.experimental.pallas import tpu as pltpu
from jax.experimental.pallas import tpu_sc as plsc

N_NODES = 10000
NP = 10240
D = 128
E_REAL = 320000
EP = 327680
NC = 2
NS = 16
NW = NC * NS
L = 16
CHUNK = 64
NCH = EP // NW // CHUNK
G = 16
NG = NCH // G
NBUF = 5
ROWS_T = NP // NS
MASK_P = 1024
MPW = MASK_P // NW

_mesh = plsc.VectorSubcoreMesh(core_axis_name="c", subcore_axis_name="s")


def _wid():
    return lax.axis_index("s") * NC + lax.axis_index("c")


@functools.partial(
    pl.kernel,
    out_type=(jax.ShapeDtypeStruct((NP,), jnp.float32),
              jax.ShapeDtypeStruct((NP,), jnp.float32)),
    mesh=_mesh,
    scratch_types=[
        pltpu.VMEM((NCH, CHUNK), jnp.int32),
        pltpu.VMEM((NCH, CHUNK), jnp.float32),
        pltpu.VMEM((ROWS_T,), jnp.float32),
        pltpu.VMEM_SHARED((NP,), jnp.float32),
    ],
)
def _deg_kernel(dst_hbm, ew_hbm, degp0_hbm, degp1_hbm, dst_s, ew_s, zb, dacc):
    cid = lax.axis_index("c")
    tid = lax.axis_index("s")
    wid = _wid()
    pltpu.sync_copy(dst_hbm.at[wid], dst_s)
    pltpu.sync_copy(ew_hbm.at[wid], ew_s)

    def zbody(r, _):
        zb[pl.ds(r * L, L)] = jnp.zeros((L,), jnp.float32)
        return 0
    lax.fori_loop(0, ROWS_T // L, zbody, 0)
    pltpu.sync_copy(zb, dacc.at[pl.ds(tid * ROWS_T, ROWS_T)])
    plsc.subcore_barrier()

    def body(k, _):
        pltpu.sync_copy(ew_s.at[k], dacc.at[dst_s.at[k]], add=True)
        return 0
    lax.fori_loop(0, NCH, body, 0)
    plsc.subcore_barrier()

    @pl.when(cid == 0)
    def _():
        pltpu.sync_copy(dacc.at[pl.ds(tid * ROWS_T, ROWS_T)],
                        degp0_hbm.at[pl.ds(tid * ROWS_T, ROWS_T)])

    @pl.when(cid == 1)
    def _():
        pltpu.sync_copy(dacc.at[pl.ds(tid * ROWS_T, ROWS_T)],
                        degp1_hbm.at[pl.ds(tid * ROWS_T, ROWS_T)])


def _dinv_body(d0_ref, d1_ref, o_ref):
    deg = d0_ref[...] + d1_ref[...] + 1.0
    o_ref[...] = lax.rsqrt(deg)


def _dinv_call(d0, d1):
    return pl.pallas_call(
        _dinv_body,
        out_shape=jax.ShapeDtypeStruct((NP // D, D), jnp.float32),
    )(d0, d1)


def _mm_body(x_ref, w_ref, d2_ref, o_ref):
    o_ref[...] = jnp.dot(x_ref[...], w_ref[...],
                         preferred_element_type=jnp.float32) * d2_ref[...]


def _mm_call(x, w, d2):
    bm = 512
    return pl.pallas_call(
        _mm_body,
        grid=(NP // bm,),
        in_specs=[pl.BlockSpec((bm, D), lambda i: (i, 0)),
                  pl.BlockSpec((D, D), lambda i: (0, 0)),
                  pl.BlockSpec((bm, D), lambda i: (i, 0))],
        out_specs=pl.BlockSpec((bm, D), lambda i: (i, 0)),
        out_shape=jax.ShapeDtypeStruct((NP, D), jnp.float32),
    )(x, w, d2)


def _layer2_body(a0_ref, a1_ref, h1p_ref, d2_ref, b_ref, w_ref, o_ref):
    z = (a0_ref[...] + a1_ref[...] + h1p_ref[...]) * d2_ref[...] + b_ref[0:1, :]
    z = jnp.maximum(z, 0.0)
    o_ref[...] = jnp.dot(z, w_ref[...],
                         preferred_element_type=jnp.float32) * d2_ref[...]


def _layer2_call(a0, a1, h1p, d2, b8, w):
    bm = 512
    return pl.pallas_call(
        _layer2_body,
        grid=(NP // bm,),
        in_specs=[pl.BlockSpec((bm, D), lambda i: (i, 0)),
                  pl.BlockSpec((bm, D), lambda i: (i, 0)),
                  pl.BlockSpec((bm, D), lambda i: (i, 0)),
                  pl.BlockSpec((bm, D), lambda i: (i, 0)),
                  pl.BlockSpec((8, D), lambda i: (0, 0)),
                  pl.BlockSpec((D, D), lambda i: (0, 0))],
        out_specs=pl.BlockSpec((bm, D), lambda i: (i, 0)),
        out_shape=jax.ShapeDtypeStruct((NP, D), jnp.float32),
    )(a0, a1, h1p, d2, b8, w)


@functools.partial(
    pl.kernel,
    out_type=(jax.ShapeDtypeStruct((NP, D), jnp.float32),
              jax.ShapeDtypeStruct((NP, D), jnp.float32)),
    mesh=_mesh,
    scratch_types=[
        pltpu.VMEM((G, CHUNK), jnp.int32),
        pltpu.VMEM((G, CHUNK), jnp.int32),
        pltpu.VMEM((G, CHUNK), jnp.float32),
        pltpu.VMEM((CHUNK, D), jnp.float32),
        pltpu.VMEM((CHUNK, D), jnp.float32),
        pltpu.VMEM((CHUNK, D), jnp.float32),
        pltpu.VMEM((CHUNK, D), jnp.float32),
        pltpu.VMEM((CHUNK, D), jnp.float32),
        pltpu.VMEM_SHARED((NP, D), jnp.float32),
        pltpu.SemaphoreType.DMA,
        pltpu.SemaphoreType.DMA,
    ],
)
def _msg_kernel(h_hbm, src_hbm, dst_hbm, ew_hbm,
                agg0_hbm, agg1_hbm,
                src_s, dst_s, ew_s, rows0, rows1, rows2, rows3, rows4,
                acc, gsem, ssem):
    cid = lax.axis_index("c")
    tid = lax.axis_index("s")
    wid = _wid()

    def zrow(r, _):
        for j in range(D // L):
            rows0[r, pl.ds(j * L, L)] = jnp.zeros((L,), jnp.float32)
        return 0
    lax.fori_loop(0, CHUNK, zrow, 0)
    zd = [pltpu.async_copy(
        rows0, acc.at[pl.ds(tid * ROWS_T + i * CHUNK, CHUNK)], ssem)
        for i in range(ROWS_T // CHUNK)]
    for d in zd:
        d.wait()
    plsc.subcore_barrier()

    bufs = (rows0, rows1, rows2, rows3, rows4)
    AHEAD = NBUF - 1

    def group(g, _):
        i1 = pltpu.async_copy(src_hbm.at[wid, pl.ds(g * G, G)], src_s, gsem)
        i2 = pltpu.async_copy(dst_hbm.at[wid, pl.ds(g * G, G)], dst_s, gsem)
        i3 = pltpu.async_copy(ew_hbm.at[wid, pl.ds(g * G, G)], ew_s, gsem)
        i1.wait()
        i2.wait()
        i3.wait()

        gd = {}
        sd = {}
        for k2 in range(AHEAD):
            gd[k2] = pltpu.async_copy(
                h_hbm.at[src_s.at[k2]], bufs[k2 % NBUF], gsem)
        for k2 in range(G):
            rb = bufs[k2 % NBUF]
            gd.pop(k2).wait()
            if k2 + AHEAD < G:
                if (k2 - 1) in sd:
                    sd.pop(k2 - 1).wait()
                gd[k2 + AHEAD] = pltpu.async_copy(
                    h_hbm.at[src_s.at[k2 + AHEAD]],
                    bufs[(k2 + AHEAD) % NBUF], gsem)

            def scale(jg, _):
                sv = ew_s[k2, pl.ds(jg * L, L)]
                for r2 in range(L):
                    sc = lax.broadcast(sv[r2], (L,))
                    r = jg * L + r2
                    for j in range(D // L):
                        sl = pl.ds(j * L, L)
                        rb[r, sl] = rb[r, sl] * sc
                return 0
            lax.fori_loop(0, CHUNK // L, scale, 0)
            sd[k2] = pltpu.async_copy(rb, acc.at[dst_s.at[k2]], ssem,
                                      add=True)
        for d in sd.values():
            d.wait()
        return 0
    lax.fori_loop(0, NG, group, 0)
    plsc.subcore_barrier()

    @pl.when(cid == 0)
    def _():
        dd = [pltpu.async_copy(
            acc.at[pl.ds(tid * ROWS_T + i * CHUNK, CHUNK)],
            agg0_hbm.at[pl.ds(tid * ROWS_T + i * CHUNK, CHUNK)], ssem)
            for i in range(ROWS_T // CHUNK)]
        for d in dd:
            d.wait()

    @pl.when(cid == 1)
    def _():
        dd = [pltpu.async_copy(
            acc.at[pl.ds(tid * ROWS_T + i * CHUNK, CHUNK)],
            agg1_hbm.at[pl.ds(tid * ROWS_T + i * CHUNK, CHUNK)], ssem)
            for i in range(ROWS_T // CHUNK)]
        for d in dd:
            d.wait()


@functools.partial(
    pl.kernel,
    out_type=(jax.ShapeDtypeStruct((MASK_P, D), jnp.float32),
              jax.ShapeDtypeStruct((MASK_P,), jnp.int32)),
    mesh=_mesh,
    scratch_types=[
        pltpu.VMEM((MPW,), jnp.int32),
        pltpu.VMEM((MPW, D), jnp.float32),
        pltpu.VMEM((MPW, D), jnp.float32),
        pltpu.VMEM((MPW, D), jnp.float32),
        pltpu.VMEM((MPW, D), jnp.float32),
        pltpu.VMEM((MPW,), jnp.int32),
        pltpu.VMEM((D,), jnp.float32),
        pltpu.SemaphoreType.DMA,
    ],
)
def _final_kernel(a0_hbm, a1_hbm, h2p_hbm, d2_hbm, b2_hbm, mask_hbm, y_hbm,
                  out_hbm, ym_hbm,
                  m_s, r0, r1, r2v, r3, yb, b2_s, sem):
    wid = _wid()
    pltpu.sync_copy(mask_hbm.at[wid], m_s)
    pltpu.sync_copy(b2_hbm, b2_s)
    g1 = pltpu.async_copy(a0_hbm.at[m_s], r0, sem)
    g2 = pltpu.async_copy(a1_hbm.at[m_s], r1, sem)
    g3 = pltpu.async_copy(h2p_hbm.at[m_s], r2v, sem)
    g4 = pltpu.async_copy(d2_hbm.at[m_s], r3, sem)
    g5 = pltpu.async_copy(y_hbm.at[m_s], yb, sem)
    g1.wait()
    g2.wait()
    g3.wait()
    g4.wait()
    g5.wait()

    def body(r, _):
        for j in range(D // L):
            sl = pl.ds(j * L, L)
            r0[r, sl] = (r0[r, sl] + r1[r, sl] + r2v[r, sl]) * r3[r, sl] \
                + b2_s[sl]
        return 0
    lax.fori_loop(0, MPW, body, 0)
    pltpu.sync_copy(r0, out_hbm.at[pl.ds(wid * MPW, MPW)])
    pltpu.sync_copy(yb, ym_hbm.at[pl.ds(wid * MPW, MPW)])


def kernel(x, edge_index, edge_weight, mask_idx, y, W1, b1, W2, b2):
    src = edge_index[0].astype(jnp.int32)
    dst = edge_index[1].astype(jnp.int32)
    ew = edge_weight.astype(jnp.float32)

    srcp = jnp.pad(src, (0, EP - E_REAL)).reshape(NW, NCH, CHUNK)
    dstp = jnp.pad(dst, (0, EP - E_REAL)).reshape(NW, NCH, CHUNK)
    ewp = jnp.pad(ew, (0, EP - E_REAL)).reshape(NW, NCH, CHUNK)

    x_p = jnp.pad(x, ((0, NP - N_NODES), (0, 0)))
    mask_p = jnp.pad(mask_idx.astype(jnp.int32),
                     (0, MASK_P - mask_idx.shape[0])).reshape(NW, MPW)
    b1_8 = jnp.broadcast_to(b1.reshape(1, D), (8, D))

    degp0, degp1 = _deg_kernel(dstp, ewp)
    dinv2d = _dinv_call(degp0.reshape(NP // D, D), degp1.reshape(NP // D, D))
    d2 = jnp.broadcast_to(dinv2d.reshape(NP)[:, None], (NP, D))

    h1p = _mm_call(x_p, W1, d2)
    a0, a1 = _msg_kernel(h1p, srcp, dstp, ewp)
    h2p = _layer2_call(a0, a1, h1p, d2, b1_8, W2)
    c0, c1 = _msg_kernel(h2p, srcp, dstp, ewp)
    out_p, ym_p = _final_kernel(c0, c1, h2p, d2, b2, mask_p,
                                y.astype(jnp.int32))
    n_mask = mask_idx.shape[0]
    return out_p[:n_mask], ym_p[:n_mask]
